# Initial kernel scaffold; baseline (speedup 1.0000x reference)
import jax
import jax.numpy as jnp
from jax.experimental import pallas as pl


def kernel(hidden_states, gate_weight, w_gate, w_up, w_down, sh_gate, sh_up, sh_down):
    raise NotImplementedError("write your pallas kernel here")


# speedup vs baseline: 1.0353x; 1.0353x over previous
import functools

import jax
import jax.numpy as jnp
from jax.experimental import pallas as pl
from jax.experimental.pallas import tpu as pltpu

E = 64
TOPK = 2
D = 2048
DFF = 1408
NSH = 2
T = 2048

TF = 128
NT = DFF // TF


def _gating_body(x_ref, gw_ref, tw_ref, ti_ref):
    x = x_ref[...]
    logits = jax.lax.dot_general(x, gw_ref[...], (((1,), (1,)), ((), ())),
                                 preferred_element_type=jnp.float32)
    m = jnp.max(logits, axis=-1, keepdims=True)
    ex = jnp.exp(logits - m)
    scores = ex / jnp.sum(ex, axis=-1, keepdims=True)
    iota = jax.lax.broadcasted_iota(jnp.int32, scores.shape, 1)
    v1 = jnp.max(scores, axis=-1, keepdims=True)
    i1 = jnp.min(jnp.where(scores >= v1, iota, E), axis=-1, keepdims=True)
    masked = jnp.where(iota == i1, -jnp.inf, scores)
    v2 = jnp.max(masked, axis=-1, keepdims=True)
    i2 = jnp.min(jnp.where(masked >= v2, iota, E), axis=-1, keepdims=True)
    denom = v1 + v2 + 1e-20
    tw_ref[:, 0:1] = v1 / denom
    tw_ref[:, 1:2] = v2 / denom
    ti_ref[:, 0:1] = i1
    ti_ref[:, 1:2] = i2


def _gating(x, gate_weight):
    return pl.pallas_call(
        _gating_body,
        out_shape=(jax.ShapeDtypeStruct((T, TOPK), jnp.float32),
                   jax.ShapeDtypeStruct((T, TOPK), jnp.int32)),
    )(x, gate_weight)


def _moe_dense_body(x_ref, tw_ref, ti_ref, wg_ref, wu_ref, wd_ref, out_ref):
    e = pl.program_id(0)
    j = pl.program_id(1)

    @pl.when(jnp.logical_and(e == 0, j == 0))
    def _():
        out_ref[...] = jnp.zeros_like(out_ref)

    x = x_ref[...]
    g = wg_ref[0]
    u = wu_ref[0]
    d = wd_ref[0]
    hg = jax.lax.dot_general(x, g, (((1,), (1,)), ((), ())),
                             preferred_element_type=jnp.float32)
    hu = jax.lax.dot_general(x, u, (((1,), (1,)), ((), ())),
                             preferred_element_type=jnp.float32)
    act = (hg * jax.lax.logistic(hg)) * hu
    y = jax.lax.dot_general(act, d, (((1,), (1,)), ((), ())),
                            preferred_element_type=jnp.float32)
    we = jnp.sum(tw_ref[...] * (ti_ref[...] == e).astype(jnp.float32), axis=-1,
                 keepdims=True)
    out_ref[...] += we * y


def _moe_dense(x, topk_w, topk_idx, w_gate, w_up, w_down):
    return pl.pallas_call(
        _moe_dense_body,
        grid=(E, NT),
        in_specs=[
            pl.BlockSpec((T, D), lambda e, j: (0, 0)),
            pl.BlockSpec((T, TOPK), lambda e, j: (0, 0)),
            pl.BlockSpec((T, TOPK), lambda e, j: (0, 0)),
            pl.BlockSpec((1, TF, D), lambda e, j: (e, j, 0)),
            pl.BlockSpec((1, TF, D), lambda e, j: (e, j, 0)),
            pl.BlockSpec((1, D, TF), lambda e, j: (e, 0, j)),
        ],
        out_specs=pl.BlockSpec((T, D), lambda e, j: (0, 0)),
        out_shape=jax.ShapeDtypeStruct((T, D), jnp.float32),
    )(x, topk_w, topk_idx, w_gate, w_up, w_down)


def _shared_body(x_ref, sg_ref, su_ref, sd_ref, out_ref):
    j = pl.program_id(0)

    @pl.when(j == 0)
    def _():
        out_ref[...] = jnp.zeros_like(out_ref)

    x = x_ref[...]
    hg = jax.lax.dot_general(x, sg_ref[...], (((1,), (1,)), ((), ())),
                             preferred_element_type=jnp.float32)
    hu = jax.lax.dot_general(x, su_ref[...], (((1,), (1,)), ((), ())),
                             preferred_element_type=jnp.float32)
    act = (hg * jax.lax.logistic(hg)) * hu
    out_ref[...] += jax.lax.dot_general(act, sd_ref[...], (((1,), (1,)), ((), ())),
                                        preferred_element_type=jnp.float32)


def _shared_mlp(x, sh_gate, sh_up, sh_down):
    nsh_t = (DFF * NSH) // TF
    return pl.pallas_call(
        _shared_body,
        grid=(nsh_t,),
        in_specs=[
            pl.BlockSpec((T, D), lambda j: (0, 0)),
            pl.BlockSpec((TF, D), lambda j: (j, 0)),
            pl.BlockSpec((TF, D), lambda j: (j, 0)),
            pl.BlockSpec((D, TF), lambda j: (0, j)),
        ],
        out_specs=pl.BlockSpec((T, D), lambda j: (0, 0)),
        out_shape=jax.ShapeDtypeStruct((T, D), jnp.float32),
    )(x, sh_gate, sh_up, sh_down)


def kernel(hidden_states, gate_weight, w_gate, w_up, w_down, sh_gate, sh_up,
           sh_down):
    bsz, seq, h = hidden_states.shape
    x = hidden_states.reshape(-1, h)
    topk_w, topk_idx = _gating(x, gate_weight)
    out = _moe_dense(x, topk_w, topk_idx, w_gate, w_up, w_down)
    out = out + _shared_mlp(x, sh_gate, sh_up, sh_down)
    return out.reshape(bsz, seq, h)


# speedup vs baseline: 3.0257x; 2.9226x over previous
import functools

import jax
import jax.numpy as jnp
from jax import lax
from jax.experimental import pallas as pl
from jax.experimental.pallas import tpu as pltpu
from jax.experimental.pallas import tpu_sc as plsc

E = 64
TOPK = 2
D = 2048
DFF = 1408
NSH = 2
T = 2048

BM = 128
NB = 96
NPAIR = T * TOPK
NSLOT = NB * BM
NC, NS = 2, 16
NW = NC * NS
TPW = T // NW
PPW = NPAIR // NW
CH = 16
NCH = TPW // CH

TF = 128
NT = DFF // TF

@functools.cache
def _sc_mesh():
    return dict(
        mesh=plsc.VectorSubcoreMesh(
            core_axis_name="c", subcore_axis_name="s", num_cores=NC,
            num_subcores=NS),
        compiler_params=pltpu.CompilerParams(needs_layout_passes=False))


def _wid():
    return lax.axis_index("s") * NC + lax.axis_index("c")


def _gating_body(x_ref, gw_ref, tw_ref, ti_ref):
    x = x_ref[...]
    logits = jax.lax.dot_general(x, gw_ref[...], (((1,), (1,)), ((), ())),
                                 preferred_element_type=jnp.float32)
    m = jnp.max(logits, axis=-1, keepdims=True)
    ex = jnp.exp(logits - m)
    scores = ex / jnp.sum(ex, axis=-1, keepdims=True)
    iota = jax.lax.broadcasted_iota(jnp.int32, scores.shape, 1)
    v1 = jnp.max(scores, axis=-1, keepdims=True)
    i1 = jnp.min(jnp.where(scores >= v1, iota, E), axis=-1, keepdims=True)
    masked = jnp.where(iota == i1, -jnp.inf, scores)
    v2 = jnp.max(masked, axis=-1, keepdims=True)
    i2 = jnp.min(jnp.where(masked >= v2, iota, E), axis=-1, keepdims=True)
    denom = v1 + v2 + 1e-20
    tw_ref[:, 0:1] = v1 / denom
    tw_ref[:, 1:2] = v2 / denom
    ti_ref[:, 0:1] = i1
    ti_ref[:, 1:2] = i2


def _gating(x, gate_weight):
    return pl.pallas_call(
        _gating_body,
        out_shape=(jax.ShapeDtypeStruct((T, TOPK), jnp.float32),
                   jax.ShapeDtypeStruct((T, TOPK), jnp.int32)),
    )(x, gate_weight)


_IOTA16 = functools.partial(lax.broadcasted_iota, jnp.int32, (16,), 0)


def _scalars_to_vec(scalars):
    v = jnp.zeros((16,), jnp.int32)
    iota = _IOTA16()
    for l, s in enumerate(scalars):
        v = jnp.where(iota == l, jnp.full((16,), s, jnp.int32), v)
    return v


def _hist_body(pairs_hbm, hist_hbm, idx_v, hist_v, hist_s):
    w = _wid()
    pltpu.sync_copy(pairs_hbm.at[pl.ds(w * PPW, PPW)], idx_v)

    def zbody(i, carry):
        hist_s[i] = 0
        return carry

    lax.fori_loop(0, E, zbody, 0)
    for c8 in range(PPW // 16):
        v = idx_v[pl.ds(c8 * 16, 16)]
        for l in range(16):
            e = v[l]
            hist_s[e] = hist_s[e] + 1
    for cj in range(E // 16):
        hist_v[pl.ds(cj * 16, 16)] = _scalars_to_vec(
            [hist_s[cj * 16 + l] for l in range(16)])
    pltpu.sync_copy(hist_v, hist_hbm.at[w])


def _histogram(pairs):
    return pl.kernel(
        _hist_body,
        out_type=jax.ShapeDtypeStruct((NW, E), jnp.int32),
        scratch_types=[pltpu.VMEM((PPW,), jnp.int32),
                       pltpu.VMEM((E,), jnp.int32),
                       pltpu.SMEM((E,), jnp.int32)],
        **_sc_mesh(),
    )(pairs)


def _dispatch_body(pairs_hbm, x_hbm, hist_hbm, xs_hbm, dest_hbm, be_hbm,
                   allh_v, be_v, idx_v, d_v, xrow_v, idx_s, base_s, cum_s):
    w = _wid()
    pltpu.sync_copy(hist_hbm, allh_v)
    pltpu.sync_copy(pairs_hbm.at[pl.ds(w * PPW, PPW)], idx_v)
    for c8 in range(PPW // 16):
        v = idx_v[pl.ds(c8 * 16, 16)]
        for l in range(16):
            idx_s[c8 * 16 + l] = v[l]

    carry = jnp.int32(0)
    for cj in range(4):
        sl = pl.ds(cj * 16, 16)
        pref = jnp.zeros((16,), jnp.int32)
        acc = jnp.zeros((16,), jnp.int32)
        for wp in range(NW):
            row = allh_v[wp, sl]
            pref = pref + jnp.where(jnp.int32(wp) < w, row,
                                    jnp.zeros_like(row))
            acc = acc + row
        nb = (acc + (BM - 1)) // BM
        cum_incl = plsc.cumsum(nb)
        pad_off = (carry + (cum_incl - nb)) * BM
        base = pad_off + pref
        cum = carry + cum_incl
        for l in range(16):
            base_s[cj * 16 + l] = base[l]
            cum_s[cj * 16 + l] = cum[l]
        carry = carry + jnp.sum(nb)

    total_blocks = carry
    last_e = jnp.int32(0)

    def lbody(e, acc):
        return acc + jnp.where(cum_s[e] < total_blocks, 1, 0)

    last_e = lax.fori_loop(0, E, lbody, last_e)

    @pl.when(w == 0)
    def _():
        for cb in range(NB // 16):
            bvec = _IOTA16() + cb * 16

            def ebody(e, cnt):
                ce = cum_s[e]
                return cnt + (ce <= bvec).astype(jnp.int32)

            cnt = lax.fori_loop(0, E, ebody, jnp.zeros((16,), jnp.int32))
            enc = jnp.where(bvec >= total_blocks, last_e + E, cnt)
            be_v[pl.ds(cb * 16, 16)] = enc
        pltpu.sync_copy(be_v, be_hbm)

    for c in range(NCH):
        pltpu.sync_copy(x_hbm.at[pl.ds(w * TPW + c * CH, CH)], xrow_v)
        d0, d1 = [], []
        for tl in range(CH):
            p = (c * CH + tl) * 2
            e0 = idx_s[p]
            pos0 = base_s[e0]
            base_s[e0] = pos0 + 1
            d0.append(pos0)
            e1 = idx_s[p + 1]
            pos1 = base_s[e1]
            base_s[e1] = pos1 + 1
            d1.append(pos1)
        d_v[0, :] = _scalars_to_vec(d0)
        d_v[1, :] = _scalars_to_vec(d1)
        pltpu.sync_copy(xrow_v, xs_hbm.at[d_v.at[0]])
        pltpu.sync_copy(xrow_v, xs_hbm.at[d_v.at[1]])
        pltpu.sync_copy(d_v.at[0], dest_hbm.at[0, w, c])
        pltpu.sync_copy(d_v.at[1], dest_hbm.at[1, w, c])


def _dispatch(pairs, x, hist):
    return pl.kernel(
        _dispatch_body,
        out_type=(jax.ShapeDtypeStruct((NSLOT, D), jnp.float32),
                  jax.ShapeDtypeStruct((TOPK, NW, NCH, CH), jnp.int32),
                  jax.ShapeDtypeStruct((NB,), jnp.int32)),
        scratch_types=[pltpu.VMEM((NW, E), jnp.int32),
                       pltpu.VMEM((NB,), jnp.int32),
                       pltpu.VMEM((PPW,), jnp.int32),
                       pltpu.VMEM((TOPK, CH), jnp.int32),
                       pltpu.VMEM((CH, D), jnp.float32),
                       pltpu.SMEM((PPW,), jnp.int32),
                       pltpu.SMEM((E,), jnp.int32),
                       pltpu.SMEM((E,), jnp.int32)],
        **_sc_mesh(),
    )(pairs, x, hist)


def _gemm_body(be_ref, xs_ref, wg_ref, wu_ref, wd_ref, out_ref):
    b = pl.program_id(0)
    j = pl.program_id(1)

    @pl.when(be_ref[b] < E)
    def _():
        x = xs_ref[...]
        hg = jax.lax.dot_general(x, wg_ref[0], (((1,), (1,)), ((), ())),
                                 preferred_element_type=jnp.float32)
        hu = jax.lax.dot_general(x, wu_ref[0], (((1,), (1,)), ((), ())),
                                 preferred_element_type=jnp.float32)
        act = (hg * jax.lax.logistic(hg)) * hu
        y = jax.lax.dot_general(act, wd_ref[0], (((1,), (1,)), ((), ())),
                                preferred_element_type=jnp.float32)

        @pl.when(j == 0)
        def _():
            out_ref[...] = y

        @pl.when(j > 0)
        def _():
            out_ref[...] += y


def _grouped_gemm(block_expert, xs, w_gate, w_up, w_down):
    grid_spec = pltpu.PrefetchScalarGridSpec(
        num_scalar_prefetch=1,
        grid=(NB, NT),
        in_specs=[
            pl.BlockSpec((BM, D), lambda b, j, be: (b, 0)),
            pl.BlockSpec((1, TF, D), lambda b, j, be: (be[b] % E, j, 0)),
            pl.BlockSpec((1, TF, D), lambda b, j, be: (be[b] % E, j, 0)),
            pl.BlockSpec((1, D, TF), lambda b, j, be: (be[b] % E, 0, j)),
        ],
        out_specs=pl.BlockSpec((BM, D), lambda b, j, be: (b, 0)),
    )
    return pl.pallas_call(
        _gemm_body,
        grid_spec=grid_spec,
        out_shape=jax.ShapeDtypeStruct((NSLOT, D), jnp.float32),
    )(block_expert, xs, w_gate, w_up, w_down)


def _gathery_body(dest_hbm, os_hbm, y_hbm, gidx_v, grow_v):
    w = _wid()
    for k in range(TOPK):
        for c in range(NCH):
            pltpu.sync_copy(dest_hbm.at[k, w, c], gidx_v)
            pltpu.sync_copy(os_hbm.at[gidx_v], grow_v)
            pltpu.sync_copy(grow_v,
                            y_hbm.at[k, pl.ds(w * TPW + c * CH, CH)])


def _gather_y(dest, out_sorted):
    return pl.kernel(
        _gathery_body,
        out_type=jax.ShapeDtypeStruct((TOPK, T, D), jnp.float32),
        scratch_types=[pltpu.VMEM((CH,), jnp.int32),
                       pltpu.VMEM((CH, D), jnp.float32)],
        **_sc_mesh(),
    )(dest, out_sorted)


def _shared_body(x_ref, sg_ref, su_ref, sd_ref, out_ref):
    j = pl.program_id(0)

    @pl.when(j == 0)
    def _():
        out_ref[...] = jnp.zeros_like(out_ref)

    x = x_ref[...]
    hg = jax.lax.dot_general(x, sg_ref[...], (((1,), (1,)), ((), ())),
                             preferred_element_type=jnp.float32)
    hu = jax.lax.dot_general(x, su_ref[...], (((1,), (1,)), ((), ())),
                             preferred_element_type=jnp.float32)
    act = (hg * jax.lax.logistic(hg)) * hu
    out_ref[...] += jax.lax.dot_general(act, sd_ref[...],
                                        (((1,), (1,)), ((), ())),
                                        preferred_element_type=jnp.float32)


def _shared_mlp(x, sh_gate, sh_up, sh_down):
    nsh_t = (DFF * NSH) // TF
    return pl.pallas_call(
        _shared_body,
        grid=(nsh_t,),
        in_specs=[
            pl.BlockSpec((T, D), lambda j: (0, 0)),
            pl.BlockSpec((TF, D), lambda j: (j, 0)),
            pl.BlockSpec((TF, D), lambda j: (j, 0)),
            pl.BlockSpec((D, TF), lambda j: (0, j)),
        ],
        out_specs=pl.BlockSpec((T, D), lambda j: (0, 0)),
        out_shape=jax.ShapeDtypeStruct((T, D), jnp.float32),
    )(x, sh_gate, sh_up, sh_down)


def _final_body(sh_ref, y_ref, tw_ref, out_ref):
    w0 = tw_ref[:, 0:1]
    w1 = tw_ref[:, 1:2]
    out_ref[...] = sh_ref[...] + w0 * y_ref[0] + w1 * y_ref[1]


def _final(sh_out, y, topk_w):
    tt = 4
    tb = T // tt
    return pl.pallas_call(
        _final_body,
        grid=(tt,),
        in_specs=[
            pl.BlockSpec((tb, D), lambda i: (i, 0)),
            pl.BlockSpec((TOPK, tb, D), lambda i: (0, i, 0)),
            pl.BlockSpec((tb, TOPK), lambda i: (i, 0)),
        ],
        out_specs=pl.BlockSpec((tb, D), lambda i: (i, 0)),
        out_shape=jax.ShapeDtypeStruct((T, D), jnp.float32),
    )(sh_out, y, topk_w)


def kernel(hidden_states, gate_weight, w_gate, w_up, w_down, sh_gate, sh_up,
           sh_down):
    bsz, seq, h = hidden_states.shape
    x = hidden_states.reshape(-1, h)
    topk_w, topk_idx = _gating(x, gate_weight)
    pairs = topk_idx.reshape(NPAIR)
    hist = _histogram(pairs)
    xs, dest, block_expert = _dispatch(pairs, x, hist)
    out_sorted = _grouped_gemm(block_expert, xs, w_gate, w_up, w_down)
    y = _gather_y(dest, out_sorted)
    sh_out = _shared_mlp(x, sh_gate, sh_up, sh_down)
    out = _final(sh_out, y, topk_w)
    return out.reshape(bsz, seq, h)
